# baseline (speedup 1.0000x reference)
import dataclasses
import functools

import jax
import jax.numpy as jnp
from jax import lax
from jax.experimental import pallas as pl
from jax.experimental.pallas import tpu as pltpu
from jax.experimental.pallas import tpu_sc as plsc

NUM_CORES = 2
NUM_SUBCORES = 16
NUM_TILES = NUM_CORES * NUM_SUBCORES
BATCHES_PER_TILE = 2
CHUNK_V = 32
CHUNK_M = 64
LANES = 16


def _sc_cmask(n_batch, n_vis, n_tok, h):
    vslots = n_tok // CHUNK_V
    mslots = n_tok // CHUNK_M
    mesh = plsc.VectorSubcoreMesh(core_axis_name="c", subcore_axis_name="s")
    cp = pltpu.CompilerParams()
    if "needs_layout_passes" in pltpu.CompilerParams.__dataclass_fields__:
        cp = dataclasses.replace(cp, needs_layout_passes=False)

    @functools.partial(
        pl.kernel,
        out_type=jax.ShapeDtypeStruct((n_batch * n_tok, h), jnp.float32),
        mesh=mesh,
        compiler_params=cp,
        scratch_types=[
            pltpu.VMEM((n_tok,), jnp.int32),
            pltpu.VMEM((n_tok,), jnp.int32),
            pltpu.VMEM((n_tok,), jnp.int32),
            pltpu.VMEM((LANES,), jnp.int32),
            pltpu.VMEM((CHUNK_M, h), jnp.float32),
            pltpu.VMEM((2, CHUNK_V, h), jnp.float32),
            pltpu.VMEM((BATCHES_PER_TILE * vslots, CHUNK_V), jnp.int32),
            pltpu.VMEM((BATCHES_PER_TILE * vslots, CHUNK_V), jnp.int32),
            pltpu.VMEM((BATCHES_PER_TILE * mslots, CHUNK_M), jnp.int32),
            pltpu.SemaphoreType.DMA,
            pltpu.SemaphoreType.DMA,
            pltpu.SemaphoreType.DMA,
        ],
    )
    def k(inp_hbm, mst_hbm, vpos_hbm, src_hbm, mpos_hbm, cnt_hbm, out_hbm,
          vpos_v, src_v, mpos_v, cnt_v, mstblk_v, rows_v,
          vsrc2d, vdst2d, mdst2d, sem_m, sem_v0, sem_v1):
        wid = lax.axis_index("s") * NUM_CORES + lax.axis_index("c")

        pltpu.sync_copy(vpos_hbm, vpos_v)
        pltpu.sync_copy(src_hbm, src_v)
        pltpu.sync_copy(mpos_hbm, mpos_v)
        pltpu.sync_copy(cnt_hbm, cnt_v)

        lane = lax.iota(jnp.int32, LANES)
        cvec = cnt_v[pl.ds(0, LANES)]
        kv = jnp.sum(jnp.where(lane == 0, cvec, 0))
        km = jnp.sum(jnp.where(lane == 1, cvec, 0))

        pltpu.sync_copy(mst_hbm, mstblk_v)

        sems_v = (sem_v0, sem_v1)
        for nb in range(BATCHES_PER_TILE):
            b = wid * BATCHES_PER_TILE + nb
            out_off = b * n_tok
            in_off = b * n_vis

            @pl.loop(0, mslots)
            def _(s, nb=nb, out_off=out_off):
                @pl.when(s * CHUNK_M < km)
                def _():
                    row = nb * mslots + s
                    for g in range(CHUNK_M // LANES):
                        pos = mpos_v[pl.ds(s * CHUNK_M + g * LANES, LANES)]
                        mdst2d[row, pl.ds(g * LANES, LANES)] = pos + out_off
                    pltpu.async_copy(mstblk_v, out_hbm.at[mdst2d.at[row]], sem_m)

            @pl.loop(0, vslots, step=2)
            def _(s0, nb=nb, out_off=out_off, in_off=in_off):
                for p in range(2):
                    s = s0 + p

                    @pl.when(jnp.logical_and(s * CHUNK_V < kv, s >= 2))
                    def _(p=p, s=s):
                        pltpu.make_async_copy(
                            rows_v.at[p], out_hbm.at[pl.ds(0, CHUNK_V)], sems_v[p]
                        ).wait()

                    @pl.when(s * CHUNK_V < kv)
                    def _(p=p, s=s):
                        row = nb * vslots + s
                        for g in range(CHUNK_V // LANES):
                            sl = pl.ds(s * CHUNK_V + g * LANES, LANES)
                            vsrc2d[row, pl.ds(g * LANES, LANES)] = src_v[sl] + in_off
                            vdst2d[row, pl.ds(g * LANES, LANES)] = vpos_v[sl] + out_off
                        pltpu.sync_copy(inp_hbm.at[vsrc2d.at[row]], rows_v.at[p])
                        pltpu.async_copy(rows_v.at[p], out_hbm.at[vdst2d.at[row]], sems_v[p])

            for p in range(2):
                @pl.when(p * CHUNK_V < kv)
                def _(p=p):
                    pltpu.make_async_copy(
                        rows_v.at[p], out_hbm.at[pl.ds(0, CHUNK_V)], sems_v[p]
                    ).wait()

        @pl.loop(0, mslots)
        def _(s):
            @pl.when(s * CHUNK_M < km)
            def _():
                for _ in range(BATCHES_PER_TILE):
                    pltpu.make_async_copy(
                        mstblk_v, out_hbm.at[pl.ds(0, CHUNK_M)], sem_m
                    ).wait()

    return k


def kernel(inputs, mask_indices, un_masked_indices, mst):
    b, n_vis, h = inputs.shape
    m = mask_indices.shape[0]
    n_tok = m + n_vis

    idx = jnp.concatenate([mask_indices, un_masked_indices]).astype(jnp.int32)
    is_mst = idx < m
    ar = jnp.arange(n_tok, dtype=jnp.int32)
    kv = jnp.sum((~is_mst).astype(jnp.int32))
    km = n_tok - kv
    rank_v = jnp.cumsum((~is_mst).astype(jnp.int32)) - 1
    rank_m = jnp.cumsum(is_mst.astype(jnp.int32)) - 1
    perm_v = jnp.zeros((n_tok,), jnp.int32).at[
        jnp.where(is_mst, n_tok, rank_v)].set(ar, mode="drop")
    perm_m = jnp.zeros((n_tok,), jnp.int32).at[
        jnp.where(is_mst, rank_m, n_tok)].set(ar, mode="drop")
    selv = jnp.where(ar < kv, ar, ar % jnp.maximum(kv, 1))
    selm = jnp.where(ar < km, ar, ar % jnp.maximum(km, 1))
    vis_pos = perm_v[selv]
    mst_pos = perm_m[selm]
    src_loc = jnp.clip(idx[vis_pos] - m, 0, n_vis - 1)
    cnt = jnp.zeros((LANES,), jnp.int32).at[0].set(kv).at[1].set(km)

    out = _sc_cmask(b, n_vis, n_tok, h)(
        inputs.reshape(b * n_vis, h),
        jnp.broadcast_to(mst.reshape(1, h).astype(inputs.dtype), (CHUNK_M, h)),
        vis_pos, src_loc, mst_pos, cnt,
    )
    return out.reshape(b, n_tok, h)


# speedup vs baseline: 1.1302x; 1.1302x over previous
import dataclasses
import functools

import jax
import jax.numpy as jnp
from jax import lax
from jax.experimental import pallas as pl
from jax.experimental.pallas import tpu as pltpu
from jax.experimental.pallas import tpu_sc as plsc

NUM_CORES = 2
NUM_SUBCORES = 16
NUM_TILES = NUM_CORES * NUM_SUBCORES
BATCHES_PER_TILE = 2
CHUNK_V = 32
CHUNK_M = 64
LANES = 16


def _sc_cmask(n_batch, n_vis, n_tok, h):
    vslots = n_tok // CHUNK_V
    mslots = n_tok // CHUNK_M
    mesh = plsc.VectorSubcoreMesh(core_axis_name="c", subcore_axis_name="s")
    cp = pltpu.CompilerParams()
    if "needs_layout_passes" in pltpu.CompilerParams.__dataclass_fields__:
        cp = dataclasses.replace(cp, needs_layout_passes=False)

    @functools.partial(
        pl.kernel,
        out_type=jax.ShapeDtypeStruct((n_batch * n_tok, h), jnp.float32),
        mesh=mesh,
        compiler_params=cp,
        scratch_types=[
            pltpu.VMEM((n_tok,), jnp.int32),
            pltpu.VMEM((n_tok,), jnp.int32),
            pltpu.VMEM((n_tok,), jnp.int32),
            pltpu.VMEM((LANES,), jnp.int32),
            pltpu.VMEM((CHUNK_M, h), jnp.float32),
            pltpu.VMEM((2, CHUNK_V, h), jnp.float32),
            pltpu.VMEM((BATCHES_PER_TILE * vslots, CHUNK_V), jnp.int32),
            pltpu.VMEM((BATCHES_PER_TILE * vslots, CHUNK_V), jnp.int32),
            pltpu.VMEM((BATCHES_PER_TILE * mslots, CHUNK_M), jnp.int32),
            pltpu.SemaphoreType.DMA,
            pltpu.SemaphoreType.DMA,
            pltpu.SemaphoreType.DMA,
        ],
    )
    def k(inp_hbm, mst_hbm, vpos_hbm, src_hbm, mpos_hbm, cnt_hbm, out_hbm,
          vpos_v, src_v, mpos_v, cnt_v, mstblk_v, rows_v,
          vsrc2d, vdst2d, mdst2d, sem_m, sem_v0, sem_v1):
        wid = lax.axis_index("s") * NUM_CORES + lax.axis_index("c")

        pltpu.sync_copy(vpos_hbm.at[wid], vpos_v)
        pltpu.sync_copy(src_hbm.at[wid], src_v)
        pltpu.sync_copy(mpos_hbm.at[wid], mpos_v)
        pltpu.sync_copy(cnt_hbm.at[wid], cnt_v)

        lane = lax.iota(jnp.int32, LANES)
        cvec = cnt_v[pl.ds(0, LANES)]
        kv = jnp.sum(jnp.where(lane == 0, cvec, 0))
        km = jnp.sum(jnp.where(lane == 1, cvec, 0))

        pltpu.sync_copy(mst_hbm.at[wid], mstblk_v)

        sems_v = (sem_v0, sem_v1)
        for nb in range(BATCHES_PER_TILE):
            b = wid * BATCHES_PER_TILE + nb
            out_off = b * n_tok
            in_off = b * n_vis

            @pl.loop(0, mslots)
            def _(s, nb=nb, out_off=out_off):
                @pl.when(s * CHUNK_M < km)
                def _():
                    row = nb * mslots + s
                    for g in range(CHUNK_M // LANES):
                        pos = mpos_v[pl.ds(s * CHUNK_M + g * LANES, LANES)]
                        mdst2d[row, pl.ds(g * LANES, LANES)] = pos + out_off
                    pltpu.async_copy(mstblk_v, out_hbm.at[mdst2d.at[row]], sem_m)

            @pl.loop(0, vslots, step=2)
            def _(s0, nb=nb, out_off=out_off, in_off=in_off):
                for p in range(2):
                    s = s0 + p

                    @pl.when(jnp.logical_and(s * CHUNK_V < kv, s >= 2))
                    def _(p=p, s=s):
                        pltpu.make_async_copy(
                            rows_v.at[p], out_hbm.at[pl.ds(0, CHUNK_V)], sems_v[p]
                        ).wait()

                    @pl.when(s * CHUNK_V < kv)
                    def _(p=p, s=s):
                        row = nb * vslots + s
                        for g in range(CHUNK_V // LANES):
                            sl = pl.ds(s * CHUNK_V + g * LANES, LANES)
                            vsrc2d[row, pl.ds(g * LANES, LANES)] = src_v[sl] + in_off
                            vdst2d[row, pl.ds(g * LANES, LANES)] = vpos_v[sl] + out_off
                        pltpu.sync_copy(inp_hbm.at[vsrc2d.at[row]], rows_v.at[p])
                        pltpu.async_copy(rows_v.at[p], out_hbm.at[vdst2d.at[row]], sems_v[p])

            for p in range(2):
                @pl.when(p * CHUNK_V < kv)
                def _(p=p):
                    pltpu.make_async_copy(
                        rows_v.at[p], out_hbm.at[pl.ds(0, CHUNK_V)], sems_v[p]
                    ).wait()

        @pl.loop(0, mslots)
        def _(s):
            @pl.when(s * CHUNK_M < km)
            def _():
                for _ in range(BATCHES_PER_TILE):
                    pltpu.make_async_copy(
                        mstblk_v, out_hbm.at[pl.ds(0, CHUNK_M)], sem_m
                    ).wait()

    return k


def kernel(inputs, mask_indices, un_masked_indices, mst):
    b, n_vis, h = inputs.shape
    m = mask_indices.shape[0]
    n_tok = m + n_vis

    idx = jnp.concatenate([mask_indices, un_masked_indices]).astype(jnp.int32)
    is_mst = idx < m
    ar = jnp.arange(n_tok, dtype=jnp.int32)
    kv = jnp.sum((~is_mst).astype(jnp.int32))
    km = n_tok - kv
    rank_v = jnp.cumsum((~is_mst).astype(jnp.int32)) - 1
    rank_m = jnp.cumsum(is_mst.astype(jnp.int32)) - 1
    perm_v = jnp.zeros((n_tok,), jnp.int32).at[
        jnp.where(is_mst, n_tok, rank_v)].set(ar, mode="drop")
    perm_m = jnp.zeros((n_tok,), jnp.int32).at[
        jnp.where(is_mst, rank_m, n_tok)].set(ar, mode="drop")
    selv = jnp.where(ar < kv, ar, ar % jnp.maximum(kv, 1))
    selm = jnp.where(ar < km, ar, ar % jnp.maximum(km, 1))
    vis_pos = perm_v[selv]
    mst_pos = perm_m[selm]
    src_loc = jnp.clip(idx[vis_pos] - m, 0, n_vis - 1)
    cnt = jnp.zeros((LANES,), jnp.int32).at[0].set(kv).at[1].set(km)

    def per_tile(x):
        return jnp.broadcast_to(x[None], (NUM_TILES,) + x.shape)

    out = _sc_cmask(b, n_vis, n_tok, h)(
        inputs.reshape(b * n_vis, h),
        per_tile(jnp.broadcast_to(mst.reshape(1, h).astype(inputs.dtype), (CHUNK_M, h))),
        per_tile(vis_pos), per_tile(src_loc), per_tile(mst_pos), per_tile(cnt),
    )
    return out.reshape(b, n_tok, h)


# speedup vs baseline: 1.2947x; 1.1455x over previous
import dataclasses
import functools

import jax
import jax.numpy as jnp
from jax import lax
from jax.experimental import pallas as pl
from jax.experimental.pallas import tpu as pltpu
from jax.experimental.pallas import tpu_sc as plsc

NUM_CORES = 2
NUM_SUBCORES = 16
NUM_TILES = NUM_CORES * NUM_SUBCORES
BATCHES_PER_TILE = 2
CHUNK_V = 32
CHUNK_M = 64
LANES = 16


def _sc_cmask(n_batch, n_vis, n_tok, h, m):
    vslots = n_tok // CHUNK_V
    mslots = n_tok // CHUNK_M
    mesh = plsc.VectorSubcoreMesh(core_axis_name="c", subcore_axis_name="s")
    cp = pltpu.CompilerParams()
    if "needs_layout_passes" in pltpu.CompilerParams.__dataclass_fields__:
        cp = dataclasses.replace(cp, needs_layout_passes=False)

    @functools.partial(
        pl.kernel,
        out_type=jax.ShapeDtypeStruct((n_batch * n_tok, h), jnp.float32),
        mesh=mesh,
        compiler_params=cp,
        scratch_types=[
            pltpu.VMEM((n_tok,), jnp.int32),
            pltpu.VMEM((n_tok + CHUNK_V,), jnp.int32),
            pltpu.VMEM((n_tok + CHUNK_V,), jnp.int32),
            pltpu.VMEM((n_tok + CHUNK_M,), jnp.int32),
            pltpu.VMEM((CHUNK_M, h), jnp.float32),
            pltpu.VMEM((2, CHUNK_V, h), jnp.float32),
            pltpu.VMEM((BATCHES_PER_TILE * vslots, CHUNK_V), jnp.int32),
            pltpu.VMEM((BATCHES_PER_TILE * vslots, CHUNK_V), jnp.int32),
            pltpu.VMEM((BATCHES_PER_TILE * mslots, CHUNK_M), jnp.int32),
            pltpu.SemaphoreType.DMA,
            pltpu.SemaphoreType.DMA,
            pltpu.SemaphoreType.DMA,
        ],
    )
    def k(inp_hbm, mst_hbm, idx_hbm, out_hbm,
          idx_v, vpos_v, src_v, mpos_v, mstblk_v, rows_v,
          vsrc2d, vdst2d, mdst2d, sem_m, sem_v0, sem_v1):
        wid = lax.axis_index("s") * NUM_CORES + lax.axis_index("c")

        pltpu.sync_copy(idx_hbm.at[wid], idx_v)
        pltpu.sync_copy(mst_hbm.at[wid], mstblk_v)

        lane = lax.iota(jnp.int32, LANES)

        def compact(g, counts):
            kv, km = counts
            v = idx_v[pl.ds(g * LANES, LANES)]
            ar_vec = g * LANES + lane
            vism = v >= m
            ones_v = vism.astype(jnp.int32)
            cs_v = jnp.cumsum(ones_v)
            pos_v = kv + cs_v - 1
            plsc.store_scatter(vpos_v, [pos_v], ar_vec, mask=vism)
            plsc.store_scatter(src_v, [pos_v], v - m, mask=vism)
            cs_m = jnp.cumsum(1 - ones_v)
            pos_m = km + cs_m - 1
            plsc.store_scatter(mpos_v, [pos_m], ar_vec, mask=~vism)
            return kv + jnp.max(cs_v), km + jnp.max(cs_m)

        kv, km = lax.fori_loop(0, n_tok // LANES, compact, (0, 0))

        vd = jnp.broadcast_to(jnp.maximum(kv - 1, 0), (LANES,))
        vdup = plsc.load_gather(vpos_v, [vd])
        sdup = plsc.load_gather(src_v, [vd])
        md = jnp.broadcast_to(jnp.maximum(km - 1, 0), (LANES,))
        mdup = plsc.load_gather(mpos_v, [md])
        for t in range(CHUNK_V // LANES):
            plsc.store_scatter(vpos_v, [kv + t * LANES + lane], vdup)
            plsc.store_scatter(src_v, [kv + t * LANES + lane], sdup)
        for t in range(CHUNK_M // LANES):
            plsc.store_scatter(mpos_v, [km + t * LANES + lane], mdup)

        sems_v = (sem_v0, sem_v1)
        for nb in range(BATCHES_PER_TILE):
            b = wid * BATCHES_PER_TILE + nb
            out_off = b * n_tok
            in_off = b * n_vis

            @pl.loop(0, mslots)
            def _(s, nb=nb, out_off=out_off):
                @pl.when(s * CHUNK_M < km)
                def _():
                    row = nb * mslots + s
                    for g in range(CHUNK_M // LANES):
                        pos = mpos_v[pl.ds(s * CHUNK_M + g * LANES, LANES)]
                        mdst2d[row, pl.ds(g * LANES, LANES)] = pos + out_off
                    pltpu.async_copy(mstblk_v, out_hbm.at[mdst2d.at[row]], sem_m)

            @pl.loop(0, vslots, step=2)
            def _(s0, nb=nb, out_off=out_off, in_off=in_off):
                for p in range(2):
                    s = s0 + p

                    @pl.when(jnp.logical_and(s * CHUNK_V < kv, s >= 2))
                    def _(p=p, s=s):
                        pltpu.make_async_copy(
                            rows_v.at[p], out_hbm.at[pl.ds(0, CHUNK_V)], sems_v[p]
                        ).wait()

                    @pl.when(s * CHUNK_V < kv)
                    def _(p=p, s=s):
                        row = nb * vslots + s
                        for g in range(CHUNK_V // LANES):
                            sl = pl.ds(s * CHUNK_V + g * LANES, LANES)
                            vsrc2d[row, pl.ds(g * LANES, LANES)] = src_v[sl] + in_off
                            vdst2d[row, pl.ds(g * LANES, LANES)] = vpos_v[sl] + out_off
                        pltpu.sync_copy(inp_hbm.at[vsrc2d.at[row]], rows_v.at[p])
                        pltpu.async_copy(rows_v.at[p], out_hbm.at[vdst2d.at[row]], sems_v[p])

            for p in range(2):
                @pl.when(p * CHUNK_V < kv)
                def _(p=p):
                    pltpu.make_async_copy(
                        rows_v.at[p], out_hbm.at[pl.ds(0, CHUNK_V)], sems_v[p]
                    ).wait()

        @pl.loop(0, mslots)
        def _(s):
            @pl.when(s * CHUNK_M < km)
            def _():
                for _ in range(BATCHES_PER_TILE):
                    pltpu.make_async_copy(
                        mstblk_v, out_hbm.at[pl.ds(0, CHUNK_M)], sem_m
                    ).wait()

    return k


def kernel(inputs, mask_indices, un_masked_indices, mst):
    b, n_vis, h = inputs.shape
    m = mask_indices.shape[0]
    n_tok = m + n_vis

    idx = jnp.concatenate([mask_indices, un_masked_indices]).astype(jnp.int32)

    def per_tile(x):
        return jnp.broadcast_to(x[None], (NUM_TILES,) + x.shape)

    out = _sc_cmask(b, n_vis, n_tok, h, m)(
        inputs.reshape(b * n_vis, h),
        per_tile(jnp.broadcast_to(mst.reshape(1, h).astype(inputs.dtype), (CHUNK_M, h))),
        per_tile(idx),
    )
    return out.reshape(b, n_tok, h)


# speedup vs baseline: 1.3840x; 1.0690x over previous
import dataclasses
import functools

import jax
import jax.numpy as jnp
from jax import lax
from jax.experimental import pallas as pl
from jax.experimental.pallas import tpu as pltpu
from jax.experimental.pallas import tpu_sc as plsc

NUM_CORES = 2
NUM_SUBCORES = 16
NUM_TILES = NUM_CORES * NUM_SUBCORES
BATCHES_PER_TILE = 2
CHUNK_V = 32
CHUNK_M = 64
LANES = 16


def _sc_cmask(n_batch, n_vis, n_tok, h, m):
    vslots = n_tok // CHUNK_V
    mslots = n_tok // CHUNK_M
    mesh = plsc.VectorSubcoreMesh(core_axis_name="c", subcore_axis_name="s")
    cp = pltpu.CompilerParams()
    if "needs_layout_passes" in pltpu.CompilerParams.__dataclass_fields__:
        cp = dataclasses.replace(cp, needs_layout_passes=False)

    @functools.partial(
        pl.kernel,
        out_type=jax.ShapeDtypeStruct((n_batch * n_tok, h), jnp.float32),
        mesh=mesh,
        compiler_params=cp,
        scratch_types=[
            pltpu.VMEM((n_tok,), jnp.int32),
            pltpu.VMEM((n_tok + CHUNK_V,), jnp.int32),
            pltpu.VMEM((n_tok + CHUNK_V,), jnp.int32),
            pltpu.VMEM((n_tok + CHUNK_M,), jnp.int32),
            pltpu.VMEM((CHUNK_M, h), jnp.float32),
            pltpu.VMEM((2, CHUNK_V, h), jnp.float32),
            pltpu.VMEM((BATCHES_PER_TILE * vslots, CHUNK_V), jnp.int32),
            pltpu.VMEM((BATCHES_PER_TILE * vslots, CHUNK_V), jnp.int32),
            pltpu.VMEM((BATCHES_PER_TILE * mslots, CHUNK_M), jnp.int32),
            pltpu.SemaphoreType.DMA,
            pltpu.SemaphoreType.DMA,
            pltpu.SemaphoreType.DMA,
        ],
    )
    def k(inp_hbm, mst_hbm, idx_hbm, out_hbm,
          idx_v, vpos_v, src_v, mpos_v, mstblk_v, rows_v,
          vsrc2d, vdst2d, mdst2d, sem_m, sem_v0, sem_v1):
        wid = lax.axis_index("s") * NUM_CORES + lax.axis_index("c")

        pltpu.sync_copy(idx_hbm.at[wid], idx_v)
        h_mstblk = pltpu.async_copy(mst_hbm.at[wid], mstblk_v, sem_m)

        lane = lax.iota(jnp.int32, LANES)

        def compact(g, counts):
            kv, km = counts
            v = idx_v[pl.ds(g * LANES, LANES)]
            ar_vec = g * LANES + lane
            vism = v >= m
            ones_v = vism.astype(jnp.int32)
            cs_v = jnp.cumsum(ones_v)
            pos_v = kv + cs_v - 1
            plsc.store_scatter(vpos_v, [pos_v], ar_vec, mask=vism)
            plsc.store_scatter(src_v, [pos_v], v - m, mask=vism)
            cs_m = jnp.cumsum(1 - ones_v)
            pos_m = km + cs_m - 1
            plsc.store_scatter(mpos_v, [pos_m], ar_vec, mask=~vism)
            return kv + jnp.max(cs_v), km + jnp.max(cs_m)

        kv, km = lax.fori_loop(0, n_tok // LANES, compact, (0, 0))

        vd = jnp.broadcast_to(jnp.maximum(kv - 1, 0), (LANES,))
        vdup = plsc.load_gather(vpos_v, [vd])
        sdup = plsc.load_gather(src_v, [vd])
        md = jnp.broadcast_to(jnp.maximum(km - 1, 0), (LANES,))
        mdup = plsc.load_gather(mpos_v, [md])
        for t in range(CHUNK_V // LANES):
            plsc.store_scatter(vpos_v, [kv + t * LANES + lane], vdup)
            plsc.store_scatter(src_v, [kv + t * LANES + lane], sdup)
        for t in range(CHUNK_M // LANES):
            plsc.store_scatter(mpos_v, [km + t * LANES + lane], mdup)

        h_mstblk.wait()

        sems_v = (sem_v0, sem_v1)
        for nb in range(BATCHES_PER_TILE):
            out_off = (wid * BATCHES_PER_TILE + nb) * n_tok

            @pl.loop(0, mslots)
            def _(s, nb=nb, out_off=out_off):
                @pl.when(s * CHUNK_M < km)
                def _():
                    row = nb * mslots + s
                    for g in range(CHUNK_M // LANES):
                        pos = mpos_v[pl.ds(s * CHUNK_M + g * LANES, LANES)]
                        mdst2d[row, pl.ds(g * LANES, LANES)] = pos + out_off
                    pltpu.async_copy(mstblk_v, out_hbm.at[mdst2d.at[row]], sem_m)

        for nb in range(BATCHES_PER_TILE):
            b = wid * BATCHES_PER_TILE + nb
            out_off = b * n_tok
            in_off = b * n_vis

            @pl.loop(0, vslots, step=2)
            def _(s0, nb=nb, out_off=out_off, in_off=in_off):
                for p in range(2):
                    s = s0 + p

                    @pl.when(jnp.logical_and(s * CHUNK_V < kv, s >= 2))
                    def _(p=p, s=s):
                        pltpu.make_async_copy(
                            rows_v.at[p], out_hbm.at[pl.ds(0, CHUNK_V)], sems_v[p]
                        ).wait()

                    @pl.when(s * CHUNK_V < kv)
                    def _(p=p, s=s):
                        row = nb * vslots + s
                        for g in range(CHUNK_V // LANES):
                            sl = pl.ds(s * CHUNK_V + g * LANES, LANES)
                            vsrc2d[row, pl.ds(g * LANES, LANES)] = src_v[sl] + in_off
                            vdst2d[row, pl.ds(g * LANES, LANES)] = vpos_v[sl] + out_off
                        pltpu.sync_copy(inp_hbm.at[vsrc2d.at[row]], rows_v.at[p])
                        pltpu.async_copy(rows_v.at[p], out_hbm.at[vdst2d.at[row]], sems_v[p])

            for p in range(2):
                @pl.when(p * CHUNK_V < kv)
                def _(p=p):
                    pltpu.make_async_copy(
                        rows_v.at[p], out_hbm.at[pl.ds(0, CHUNK_V)], sems_v[p]
                    ).wait()

        @pl.loop(0, mslots)
        def _(s):
            @pl.when(s * CHUNK_M < km)
            def _():
                for _ in range(BATCHES_PER_TILE):
                    pltpu.make_async_copy(
                        mstblk_v, out_hbm.at[pl.ds(0, CHUNK_M)], sem_m
                    ).wait()

    return k


def kernel(inputs, mask_indices, un_masked_indices, mst):
    b, n_vis, h = inputs.shape
    m = mask_indices.shape[0]
    n_tok = m + n_vis

    idx = jnp.concatenate([mask_indices, un_masked_indices]).astype(jnp.int32)

    def per_tile(x):
        return jnp.broadcast_to(x[None], (NUM_TILES,) + x.shape)

    out = _sc_cmask(b, n_vis, n_tok, h, m)(
        inputs.reshape(b * n_vis, h),
        per_tile(jnp.broadcast_to(mst.reshape(1, h).astype(inputs.dtype), (CHUNK_M, h))),
        per_tile(idx),
    )
    return out.reshape(b, n_tok, h)
